# baseline (speedup 1.0000x reference)
import functools

import jax
import jax.numpy as jnp
from jax import lax
from jax.experimental import pallas as pl
from jax.experimental.pallas import tpu as pltpu
from jax.experimental.pallas import tpu_sc as plsc

N = 10000
D = 128
E = 320000
NEG = 0.01

NC = 2
NS = 16
NW = NC * NS
K = 128
STEPS = 80
RNDS = 4
R = STEPS // RNDS
EPW = K * STEPS
EPAD = NW * EPW
NPAD = 10240
RPS = NPAD // NS

_mesh = plsc.VectorSubcoreMesh(core_axis_name="c", subcore_axis_name="s")


def _leaky(v):
    return jnp.where(v >= 0, v, NEG * v)


def _pack_edges(edge_index):
    npad = EPAD - E
    i = jnp.arange(npad, dtype=jnp.int32)
    pad_src = (i * 37) % N
    pad_dst = N + (i % (NPAD - N))
    src = jnp.concatenate([edge_index[0], pad_src]).reshape(NW, RNDS, R, K)
    dst = jnp.concatenate([edge_index[1], pad_dst]).reshape(NW, RNDS, R, K)
    return jnp.stack([src, dst], axis=3)


def _deg_partials(p1, p2, p3, ones_k, zeros_r):
    @functools.partial(
        pl.kernel,
        out_type=jax.ShapeDtypeStruct((3, NC, NPAD), jnp.float32),
        mesh=_mesh,
        scratch_types=[
            pltpu.VMEM((R, 2, K), jnp.int32),
            pltpu.VMEM((K,), jnp.float32),
            pltpu.VMEM_SHARED((NPAD,), jnp.float32),
            pltpu.VMEM_SHARED((NPAD,), jnp.float32),
            pltpu.VMEM_SHARED((NPAD,), jnp.float32),
            pltpu.SemaphoreType.DMA,
        ],
    )
    def deg_kernel(p1h, p2h, p3h, onesh, zeroh, outh, ibuf, onesv,
                   acc1, acc2, acc3, sem):
        c = lax.axis_index("c")
        s = lax.axis_index("s")
        w = s * NC + c
        accs = (acc1, acc2, acc3)
        pltpu.async_copy(onesh, onesv, sem).wait()
        for k in range(3):
            pltpu.async_copy(
                zeroh, accs[k].at[pl.ds(s * RPS, RPS)], sem).wait()
        plsc.subcore_barrier()
        for k, ph in enumerate((p1h, p2h, p3h)):
            acc = accs[k]
            for r in range(RNDS):
                pltpu.async_copy(ph.at[w, r], ibuf, sem).wait()

                @pl.loop(0, R)
                def _(j):
                    pltpu.async_copy(
                        onesv, acc.at[ibuf.at[j, 1]], sem, add=True).wait()

        plsc.subcore_barrier()
        for k in range(3):
            pltpu.async_copy(
                accs[k].at[pl.ds(s * RPS, RPS)],
                outh.at[k, c, pl.ds(s * RPS, RPS)], sem).wait()

    return deg_kernel(p1, p2, p3, ones_k, zeros_r)


def _dinv_from_partials(degp):
    def body(p_ref, o_ref):
        deg = p_ref[:, 0] + p_ref[:, 1] + 1.0
        o_ref[...] = lax.rsqrt(deg).T

    return pl.pallas_call(
        body,
        out_shape=jax.ShapeDtypeStruct((NPAD, 3), jnp.float32),
    )(degp)


def _matmul_scaled(x, wc, dinv):
    nb = 5
    rb = N // nb

    def body(x_ref, w_ref, d_ref, y1_ref, y2_ref, y3_ref):
        xw = jnp.dot(x_ref[...], w_ref[...],
                     preferred_element_type=jnp.float32,
                     precision=lax.Precision.HIGHEST)
        d = d_ref[...]
        y1_ref[...] = xw[:, :D] * d[:, 0][:, None]
        y2_ref[...] = xw[:, D:2 * D] * d[:, 1][:, None]
        y3_ref[...] = xw[:, 2 * D:] * d[:, 2][:, None]

    yspec = pl.BlockSpec((rb, D), lambda i: (i, 0))
    return pl.pallas_call(
        body,
        grid=(nb,),
        in_specs=[
            pl.BlockSpec((rb, D), lambda i: (i, 0)),
            pl.BlockSpec((D, 3 * D), lambda i: (0, 0)),
            pl.BlockSpec((rb, 3), lambda i: (i, 0)),
        ],
        out_specs=[yspec, yspec, yspec],
        out_shape=[jax.ShapeDtypeStruct((N, D), jnp.float32)] * 3,
    )(x, wc, dinv)


def _conv1(y, pk, zeros_rows):
    @functools.partial(
        pl.kernel,
        out_type=jax.ShapeDtypeStruct((NC, NPAD, D), jnp.float32),
        mesh=_mesh,
        scratch_types=[
            pltpu.VMEM((R, 2, K), jnp.int32),
            pltpu.VMEM((R, 2, K), jnp.int32),
            pltpu.VMEM((K, D), jnp.float32),
            pltpu.VMEM((K, D), jnp.float32),
            pltpu.VMEM_SHARED((NPAD, D), jnp.float32),
            pltpu.SemaphoreType.DMA,
            pltpu.SemaphoreType.DMA,
            pltpu.SemaphoreType.DMA,
            pltpu.SemaphoreType.DMA,
        ],
    )
    def conv_kernel(yh, ph, zeroh,
                    outh, ibufa, ibufb, rowa, rowb, acc,
                    gsema, gsemb, ssem, isem):
        c = lax.axis_index("c")
        s = lax.axis_index("s")
        w = s * NC + c
        ibufs = (ibufa, ibufb)
        pltpu.async_copy(
            zeroh, acc.at[pl.ds(s * RPS, RPS)], ssem).wait()
        pltpu.async_copy(ph.at[w, 0], ibufa, isem).wait()
        plsc.subcore_barrier()
        for r in range(RNDS):
            icur = ibufs[r % 2]
            if r + 1 < RNDS:
                nxt = pltpu.async_copy(
                    ph.at[w, r + 1], ibufs[(r + 1) % 2], isem)
            pltpu.async_copy(yh.at[icur.at[0, 0]], rowa, gsema)
            pltpu.async_copy(yh.at[icur.at[1, 0]], rowb, gsemb)

            @pl.loop(0, R - 2, step=2)
            def _(j):
                pltpu.make_async_copy(
                    yh.at[icur.at[0, 0]], rowa, gsema).wait()
                pltpu.async_copy(
                    rowa, acc.at[icur.at[j, 1]], ssem, add=True).wait()
                pltpu.async_copy(yh.at[icur.at[j + 2, 0]], rowa, gsema)
                pltpu.make_async_copy(
                    yh.at[icur.at[0, 0]], rowb, gsemb).wait()
                pltpu.async_copy(
                    rowb, acc.at[icur.at[j + 1, 1]], ssem, add=True
                ).wait()
                pltpu.async_copy(yh.at[icur.at[j + 3, 0]], rowb, gsemb)

            pltpu.make_async_copy(
                yh.at[icur.at[0, 0]], rowa, gsema).wait()
            pltpu.async_copy(
                rowa, acc.at[icur.at[R - 2, 1]], ssem, add=True).wait()
            pltpu.make_async_copy(
                yh.at[icur.at[0, 0]], rowb, gsemb).wait()
            pltpu.async_copy(
                rowb, acc.at[icur.at[R - 1, 1]], ssem, add=True).wait()
            if r + 1 < RNDS:
                nxt.wait()

        plsc.subcore_barrier()
        pltpu.async_copy(
            acc.at[pl.ds(s * RPS, RPS)],
            outh.at[c, pl.ds(s * RPS, RPS)], ssem).wait()

    return conv_kernel(y, pk, zeros_rows)


def _combine(ps, y1, y2, y3, dinv, bstack, avec, residuals):
    nb = 5
    rb = N // nb
    nres = len(residuals)

    def body(p1_ref, p2_ref, p3_ref, y1_ref, y2_ref, y3_ref,
             d_ref, b_ref, a_ref, *rest):
        res_refs = rest[:nres]
        o_ref = rest[nres]
        d = d_ref[...]
        b = b_ref[...]
        a = a_ref[...]
        ys = (y1_ref[...], y2_ref[...], y3_ref[...])
        p_refs = (p1_ref, p2_ref, p3_ref)
        acc = jnp.zeros((rb, D), jnp.float32)
        for k in range(3):
            agg = p_refs[k][0] + p_refs[k][1] + ys[k]
            hk = _leaky(d[:, k][:, None] * agg + b[k][None, :])
            acc = acc + a[0, k] * hk
        for j in range(nres):
            acc = acc + a[0, 3 + j] * res_refs[j][...]
        o_ref[...] = _leaky(acc)

    rspec = pl.BlockSpec((rb, D), lambda i: (i, 0))
    pspec = pl.BlockSpec((NC, rb, D), lambda i: (0, i, 0))
    return pl.pallas_call(
        body,
        grid=(nb,),
        in_specs=[
            pspec, pspec, pspec,
            rspec, rspec, rspec,
            pl.BlockSpec((rb, 3), lambda i: (i, 0)),
            pl.BlockSpec((3, D), lambda i: (0, 0)),
            pl.BlockSpec((1, 8), lambda i: (0, 0)),
        ] + [rspec] * nres,
        out_specs=rspec,
        out_shape=jax.ShapeDtypeStruct((N, D), jnp.float32),
    )(*ps, y1, y2, y3, dinv, bstack, avec, *residuals)


def _combine_mm(ps, y1, y2, y3, dinv, bstack, avec, residuals, wc):
    nb = 5
    rb = N // nb
    nres = len(residuals)

    def body(p1_ref, p2_ref, p3_ref, y1_ref, y2_ref, y3_ref,
             d_ref, b_ref, a_ref, w_ref, *rest):
        res_refs = rest[:nres]
        o_ref, ny1_ref, ny2_ref, ny3_ref = rest[nres:]
        d = d_ref[...]
        b = b_ref[...]
        a = a_ref[...]
        ys = (y1_ref[...], y2_ref[...], y3_ref[...])
        p_refs = (p1_ref, p2_ref, p3_ref)
        acc = jnp.zeros((rb, D), jnp.float32)
        for k in range(3):
            agg = p_refs[k][0] + p_refs[k][1] + ys[k]
            hk = _leaky(d[:, k][:, None] * agg + b[k][None, :])
            acc = acc + a[0, k] * hk
        for j in range(nres):
            acc = acc + a[0, 3 + j] * res_refs[j][...]
        xn = _leaky(acc)
        o_ref[...] = xn
        xw = jnp.dot(xn, w_ref[...],
                     preferred_element_type=jnp.float32,
                     precision=lax.Precision.HIGHEST)
        ny1_ref[...] = xw[:, :D] * d[:, 0][:, None]
        ny2_ref[...] = xw[:, D:2 * D] * d[:, 1][:, None]
        ny3_ref[...] = xw[:, 2 * D:] * d[:, 2][:, None]

    rspec = pl.BlockSpec((rb, D), lambda i: (i, 0))
    pspec = pl.BlockSpec((NC, rb, D), lambda i: (0, i, 0))
    return pl.pallas_call(
        body,
        grid=(nb,),
        in_specs=[
            pspec, pspec, pspec,
            rspec, rspec, rspec,
            pl.BlockSpec((rb, 3), lambda i: (i, 0)),
            pl.BlockSpec((3, D), lambda i: (0, 0)),
            pl.BlockSpec((1, 8), lambda i: (0, 0)),
            pl.BlockSpec((D, 3 * D), lambda i: (0, 0)),
        ] + [rspec] * nres,
        out_specs=[rspec, rspec, rspec, rspec],
        out_shape=[jax.ShapeDtypeStruct((N, D), jnp.float32)] * 4,
    )(*ps, y1, y2, y3, dinv, bstack, avec, wc, *residuals)


def kernel(x, edge_index_1, edge_index_2, edge_index_3,
           W1, b1, W2, b2, W3, b3, a1, a2, a3, a4):
    packed = [_pack_edges(e) for e in (edge_index_1, edge_index_2, edge_index_3)]
    ones_k = jnp.ones((K,), jnp.float32)
    zeros_r = jnp.zeros((RPS,), jnp.float32)
    zeros_rows = jnp.zeros((RPS, D), jnp.float32)
    wc = jnp.concatenate([W1, W2, W3], axis=1)
    bstack = jnp.stack([b1, b2, b3])

    degp = _deg_partials(packed[0], packed[1], packed[2], ones_k, zeros_r)
    dinv = _dinv_from_partials(degp)

    def pad8(a):
        return jnp.concatenate([a, jnp.zeros((8 - a.shape[0],), a.dtype)]
                               ).reshape(1, 8)

    xs = [x]
    res_orders = [
        [0],
        [1, 0],
        [2, 0, 1],
        [2, 0, 1, 3],
    ]
    y1, y2, y3 = _matmul_scaled(xs[-1], wc, dinv)
    for lyr, avec in enumerate((a1, a2, a3, a4)):
        ps = [_conv1(y, pk, zeros_rows)
              for y, pk in zip((y1, y2, y3), packed)]
        residuals = [xs[j] for j in res_orders[lyr]]
        if lyr < 3:
            xn, y1, y2, y3 = _combine_mm(ps, y1, y2, y3, dinv, bstack,
                                         pad8(avec), residuals, wc)
            xs.append(xn)
        else:
            xs.append(_combine(ps, y1, y2, y3, dinv, bstack,
                               pad8(avec), residuals))
    return xs[-1]


# speedup vs baseline: 1.0357x; 1.0357x over previous
import functools

import jax
import jax.numpy as jnp
from jax import lax
from jax.experimental import pallas as pl
from jax.experimental.pallas import tpu as pltpu
from jax.experimental.pallas import tpu_sc as plsc

N = 10000
D = 128
E = 320000
NEG = 0.01

NC = 2
NS = 16
NW = NC * NS
K = 128
STEPS = 80
RNDS = 4
R = STEPS // RNDS
EPW = K * STEPS
EPAD = NW * EPW
NPAD = 10240
RPS = NPAD // NS

_mesh = plsc.VectorSubcoreMesh(core_axis_name="c", subcore_axis_name="s")


def _leaky(v):
    return jnp.where(v >= 0, v, NEG * v)


def _pack_edges(edge_index):
    npad = EPAD - E
    i = jnp.arange(npad, dtype=jnp.int32)
    pad_src = (i * 37) % N
    pad_dst = N + (i % (NPAD - N))
    src = jnp.concatenate([edge_index[0], pad_src]).reshape(NW, RNDS, R, K)
    dst = jnp.concatenate([edge_index[1], pad_dst]).reshape(NW, RNDS, R, K)
    return jnp.stack([src, dst], axis=3)


def _deg_partials(p1, p2, p3, ones_k, zeros_r):
    @functools.partial(
        pl.kernel,
        out_type=jax.ShapeDtypeStruct((3, NC, NPAD), jnp.float32),
        mesh=_mesh,
        scratch_types=[
            pltpu.VMEM((R, 2, K), jnp.int32),
            pltpu.VMEM((K,), jnp.float32),
            pltpu.VMEM_SHARED((NPAD,), jnp.float32),
            pltpu.VMEM_SHARED((NPAD,), jnp.float32),
            pltpu.VMEM_SHARED((NPAD,), jnp.float32),
            pltpu.SemaphoreType.DMA,
        ],
    )
    def deg_kernel(p1h, p2h, p3h, onesh, zeroh, outh, ibuf, onesv,
                   acc1, acc2, acc3, sem):
        c = lax.axis_index("c")
        s = lax.axis_index("s")
        w = s * NC + c
        accs = (acc1, acc2, acc3)
        pltpu.async_copy(onesh, onesv, sem).wait()
        for k in range(3):
            pltpu.async_copy(
                zeroh, accs[k].at[pl.ds(s * RPS, RPS)], sem).wait()
        plsc.subcore_barrier()
        for k, ph in enumerate((p1h, p2h, p3h)):
            acc = accs[k]
            for r in range(RNDS):
                pltpu.async_copy(ph.at[w, r], ibuf, sem).wait()

                @pl.loop(0, R)
                def _(j):
                    pltpu.async_copy(
                        onesv, acc.at[ibuf.at[j, 1]], sem, add=True).wait()

        plsc.subcore_barrier()
        for k in range(3):
            pltpu.async_copy(
                accs[k].at[pl.ds(s * RPS, RPS)],
                outh.at[k, c, pl.ds(s * RPS, RPS)], sem).wait()

    return deg_kernel(p1, p2, p3, ones_k, zeros_r)


def _dinv_from_partials(degp):
    def body(p_ref, o_ref):
        deg = p_ref[:, 0] + p_ref[:, 1] + 1.0
        o_ref[...] = lax.rsqrt(deg).T

    return pl.pallas_call(
        body,
        out_shape=jax.ShapeDtypeStruct((NPAD, 3), jnp.float32),
    )(degp)


def _matmul_scaled(x, wc, dinv):
    nb = 5
    rb = N // nb

    def body(x_ref, w_ref, d_ref, y1_ref, y2_ref, y3_ref):
        xw = jnp.dot(x_ref[...], w_ref[...],
                     preferred_element_type=jnp.float32,
                     precision=lax.Precision.HIGHEST)
        d = d_ref[...]
        y1_ref[...] = xw[:, :D] * d[:, 0][:, None]
        y2_ref[...] = xw[:, D:2 * D] * d[:, 1][:, None]
        y3_ref[...] = xw[:, 2 * D:] * d[:, 2][:, None]

    yspec = pl.BlockSpec((rb, D), lambda i: (i, 0))
    return pl.pallas_call(
        body,
        grid=(nb,),
        in_specs=[
            pl.BlockSpec((rb, D), lambda i: (i, 0)),
            pl.BlockSpec((D, 3 * D), lambda i: (0, 0)),
            pl.BlockSpec((rb, 3), lambda i: (i, 0)),
        ],
        out_specs=[yspec, yspec, yspec],
        out_shape=[jax.ShapeDtypeStruct((N, D), jnp.float32)] * 3,
    )(x, wc, dinv)


def _conv3(y1, y2, y3, packed, zeros_rows):
    p1, p2, p3 = packed

    @functools.partial(
        pl.kernel,
        out_type=jax.ShapeDtypeStruct((3, NC, NPAD, D), jnp.float32),
        mesh=_mesh,
        scratch_types=[
            pltpu.VMEM((R, 2, K), jnp.int32),
            pltpu.VMEM((R, 2, K), jnp.int32),
            pltpu.VMEM((K, D), jnp.float32),
            pltpu.VMEM((K, D), jnp.float32),
            pltpu.VMEM_SHARED((NPAD, D), jnp.float32),
            pltpu.SemaphoreType.DMA,
            pltpu.SemaphoreType.DMA,
            pltpu.SemaphoreType.DMA,
            pltpu.SemaphoreType.DMA,
        ],
    )
    def conv_kernel(y1h, y2h, y3h, p1h, p2h, p3h, zeroh,
                    outh, ibufa, ibufb, rowa, rowb, acc,
                    gsema, gsemb, ssem, isem):
        c = lax.axis_index("c")
        s = lax.axis_index("s")
        w = s * NC + c
        ibufs = (ibufa, ibufb)
        for k, (yh, ph) in enumerate(((y1h, p1h), (y2h, p2h), (y3h, p3h))):
            _conv_set(k, yh, ph, zeroh, outh, ibufs, rowa, rowb, acc,
                      gsema, gsemb, ssem, isem, c, s, w)

    return conv_kernel(y1, y2, y3, p1, p2, p3, zeros_rows)


def _conv_set(k, yh, ph, zeroh, outh, ibufs, rowa, rowb, acc,
              gsema, gsemb, ssem, isem, c, s, w):
        ibufa = ibufs[0]
        pltpu.async_copy(
            zeroh, acc.at[pl.ds(s * RPS, RPS)], ssem).wait()
        pltpu.async_copy(ph.at[w, 0], ibufa, isem).wait()
        plsc.subcore_barrier()
        for r in range(RNDS):
            icur = ibufs[r % 2]
            if r + 1 < RNDS:
                nxt = pltpu.async_copy(
                    ph.at[w, r + 1], ibufs[(r + 1) % 2], isem)
            pltpu.async_copy(yh.at[icur.at[0, 0]], rowa, gsema)
            pltpu.async_copy(yh.at[icur.at[1, 0]], rowb, gsemb)

            @pl.loop(0, R - 2, step=2)
            def _(j):
                pltpu.make_async_copy(
                    yh.at[icur.at[0, 0]], rowa, gsema).wait()
                pltpu.async_copy(
                    rowa, acc.at[icur.at[j, 1]], ssem, add=True).wait()
                pltpu.async_copy(yh.at[icur.at[j + 2, 0]], rowa, gsema)
                pltpu.make_async_copy(
                    yh.at[icur.at[0, 0]], rowb, gsemb).wait()
                pltpu.async_copy(
                    rowb, acc.at[icur.at[j + 1, 1]], ssem, add=True
                ).wait()
                pltpu.async_copy(yh.at[icur.at[j + 3, 0]], rowb, gsemb)

            pltpu.make_async_copy(
                yh.at[icur.at[0, 0]], rowa, gsema).wait()
            pltpu.async_copy(
                rowa, acc.at[icur.at[R - 2, 1]], ssem, add=True).wait()
            pltpu.make_async_copy(
                yh.at[icur.at[0, 0]], rowb, gsemb).wait()
            pltpu.async_copy(
                rowb, acc.at[icur.at[R - 1, 1]], ssem, add=True).wait()
            if r + 1 < RNDS:
                nxt.wait()

        plsc.subcore_barrier()
        pltpu.async_copy(
            acc.at[pl.ds(s * RPS, RPS)],
            outh.at[k, c, pl.ds(s * RPS, RPS)], ssem).wait()
        plsc.subcore_barrier()


def _combine(p, y1, y2, y3, dinv, bstack, avec, residuals):
    nb = 5
    rb = N // nb
    nres = len(residuals)

    def body(p_ref, y1_ref, y2_ref, y3_ref,
             d_ref, b_ref, a_ref, *rest):
        res_refs = rest[:nres]
        o_ref = rest[nres]
        d = d_ref[...]
        b = b_ref[...]
        a = a_ref[...]
        ys = (y1_ref[...], y2_ref[...], y3_ref[...])
        acc = jnp.zeros((rb, D), jnp.float32)
        for k in range(3):
            agg = p_ref[k, 0] + p_ref[k, 1] + ys[k]
            hk = _leaky(d[:, k][:, None] * agg + b[k][None, :])
            acc = acc + a[0, k] * hk
        for j in range(nres):
            acc = acc + a[0, 3 + j] * res_refs[j][...]
        o_ref[...] = _leaky(acc)

    rspec = pl.BlockSpec((rb, D), lambda i: (i, 0))
    return pl.pallas_call(
        body,
        grid=(nb,),
        in_specs=[
            pl.BlockSpec((3, NC, rb, D), lambda i: (0, 0, i, 0)),
            rspec, rspec, rspec,
            pl.BlockSpec((rb, 3), lambda i: (i, 0)),
            pl.BlockSpec((3, D), lambda i: (0, 0)),
            pl.BlockSpec((1, 8), lambda i: (0, 0)),
        ] + [rspec] * nres,
        out_specs=rspec,
        out_shape=jax.ShapeDtypeStruct((N, D), jnp.float32),
    )(p, y1, y2, y3, dinv, bstack, avec, *residuals)


def _combine_mm(p, y1, y2, y3, dinv, bstack, avec, residuals, wc):
    nb = 5
    rb = N // nb
    nres = len(residuals)

    def body(p_ref, y1_ref, y2_ref, y3_ref,
             d_ref, b_ref, a_ref, w_ref, *rest):
        res_refs = rest[:nres]
        o_ref, ny1_ref, ny2_ref, ny3_ref = rest[nres:]
        d = d_ref[...]
        b = b_ref[...]
        a = a_ref[...]
        ys = (y1_ref[...], y2_ref[...], y3_ref[...])
        acc = jnp.zeros((rb, D), jnp.float32)
        for k in range(3):
            agg = p_ref[k, 0] + p_ref[k, 1] + ys[k]
            hk = _leaky(d[:, k][:, None] * agg + b[k][None, :])
            acc = acc + a[0, k] * hk
        for j in range(nres):
            acc = acc + a[0, 3 + j] * res_refs[j][...]
        xn = _leaky(acc)
        o_ref[...] = xn
        xw = jnp.dot(xn, w_ref[...],
                     preferred_element_type=jnp.float32,
                     precision=lax.Precision.HIGHEST)
        ny1_ref[...] = xw[:, :D] * d[:, 0][:, None]
        ny2_ref[...] = xw[:, D:2 * D] * d[:, 1][:, None]
        ny3_ref[...] = xw[:, 2 * D:] * d[:, 2][:, None]

    rspec = pl.BlockSpec((rb, D), lambda i: (i, 0))
    return pl.pallas_call(
        body,
        grid=(nb,),
        in_specs=[
            pl.BlockSpec((3, NC, rb, D), lambda i: (0, 0, i, 0)),
            rspec, rspec, rspec,
            pl.BlockSpec((rb, 3), lambda i: (i, 0)),
            pl.BlockSpec((3, D), lambda i: (0, 0)),
            pl.BlockSpec((1, 8), lambda i: (0, 0)),
            pl.BlockSpec((D, 3 * D), lambda i: (0, 0)),
        ] + [rspec] * nres,
        out_specs=[rspec, rspec, rspec, rspec],
        out_shape=[jax.ShapeDtypeStruct((N, D), jnp.float32)] * 4,
    )(p, y1, y2, y3, dinv, bstack, avec, wc, *residuals)


def kernel(x, edge_index_1, edge_index_2, edge_index_3,
           W1, b1, W2, b2, W3, b3, a1, a2, a3, a4):
    packed = [_pack_edges(e) for e in (edge_index_1, edge_index_2, edge_index_3)]
    ones_k = jnp.ones((K,), jnp.float32)
    zeros_r = jnp.zeros((RPS,), jnp.float32)
    zeros_rows = jnp.zeros((RPS, D), jnp.float32)
    wc = jnp.concatenate([W1, W2, W3], axis=1)
    bstack = jnp.stack([b1, b2, b3])

    degp = _deg_partials(packed[0], packed[1], packed[2], ones_k, zeros_r)
    dinv = _dinv_from_partials(degp)

    def pad8(a):
        return jnp.concatenate([a, jnp.zeros((8 - a.shape[0],), a.dtype)]
                               ).reshape(1, 8)

    xs = [x]
    res_orders = [
        [0],
        [1, 0],
        [2, 0, 1],
        [2, 0, 1, 3],
    ]
    y1, y2, y3 = _matmul_scaled(xs[-1], wc, dinv)
    for lyr, avec in enumerate((a1, a2, a3, a4)):
        p = _conv3(y1, y2, y3, packed, zeros_rows)
        residuals = [xs[j] for j in res_orders[lyr]]
        if lyr < 3:
            xn, y1, y2, y3 = _combine_mm(p, y1, y2, y3, dinv, bstack,
                                         pad8(avec), residuals, wc)
            xs.append(xn)
        else:
            xs.append(_combine(p, y1, y2, y3, dinv, bstack,
                               pad8(avec), residuals))
    return xs[-1]
